# baseline (speedup 1.0000x reference)
import functools

import jax
import jax.numpy as jnp
from jax import lax
from jax.experimental import pallas as pl
from jax.experimental.pallas import tpu as pltpu
from jax.experimental.pallas import tpu_sc as plsc

N = 10000
E = 160000
H = 256
O = 256
R = 8
B = 4

NSEG = N * R
HC = 16
NCHUNK = H // HC
NC = 2
NS = 16
CPC = NCHUNK // NC
EPT = E // NS
W = 400
NWIN = EPT // W
SEG_PT = NSEG // NS
TAB_PT = 640
ZROWS = 250


def _sc_agg_body(xc_hbm, src_hbm, seg_hbm, norm_hbm, zeros_hbm, agg_hbm,
                 table_s, agg_s, zbuf, idx_v, seg_v, norm_v, rows_v,
                 gsem):
    c = lax.axis_index("c")
    s = lax.axis_index("s")

    pltpu.sync_copy(zeros_hbm, zbuf)

    for i in range(CPC):
        hc = c * CPC + i

        for z in range(SEG_PT // ZROWS):
            pltpu.sync_copy(
                zbuf, agg_s.at[pl.ds(s * SEG_PT + z * ZROWS, ZROWS)])

        @pl.when(s < NS - 1)
        def _():
            pltpu.sync_copy(
                xc_hbm.at[hc, pl.ds(s * TAB_PT, TAB_PT), :],
                table_s.at[pl.ds(s * TAB_PT, TAB_PT)],
            )

        @pl.when(s == NS - 1)
        def _():
            pltpu.sync_copy(
                xc_hbm.at[hc, pl.ds((NS - 1) * TAB_PT, N - (NS - 1) * TAB_PT), :],
                table_s.at[pl.ds((NS - 1) * TAB_PT, N - (NS - 1) * TAB_PT)],
            )

        plsc.subcore_barrier()

        def window(w, _):
            base = s * EPT + w * W
            pltpu.sync_copy(src_hbm.at[pl.ds(base, W)], idx_v)
            pltpu.sync_copy(seg_hbm.at[pl.ds(base, W)], seg_v)
            pltpu.sync_copy(norm_hbm.at[pl.ds(base, W)], norm_v)
            pltpu.async_copy(table_s.at[idx_v], rows_v, gsem).wait()

            def scale(g, _):
                nv = norm_v[pl.ds(g * 16, 16)]
                for j in range(16):
                    e = g * 16 + j
                    rows_v[e, :] = rows_v[e, :] * nv[j]
                return 0

            lax.fori_loop(0, W // 16, scale, 0)
            pltpu.sync_copy(rows_v, agg_s.at[seg_v], add=True)
            return 0

        lax.fori_loop(0, NWIN, window, 0)
        plsc.subcore_barrier()

        pltpu.sync_copy(
            agg_s.at[pl.ds(s * SEG_PT, SEG_PT)],
            agg_hbm.at[hc, pl.ds(s * SEG_PT, SEG_PT), :],
        )


_sc_aggregate = functools.partial(
    pl.kernel,
    out_type=jax.ShapeDtypeStruct((NCHUNK, NSEG, HC), jnp.float32),
    mesh=plsc.VectorSubcoreMesh(
        core_axis_name="c", subcore_axis_name="s", num_cores=NC,
        num_subcores=NS),
    scratch_types=[
        pltpu.VMEM_SHARED((N, HC), jnp.float32),
        pltpu.VMEM_SHARED((NSEG, HC), jnp.float32),
        pltpu.VMEM((ZROWS, HC), jnp.float32),
        pltpu.VMEM((W,), jnp.int32),
        pltpu.VMEM((W,), jnp.int32),
        pltpu.VMEM((W,), jnp.float32),
        pltpu.VMEM((W, HC), jnp.float32),
        pltpu.SemaphoreType.DMA,
    ],
    compiler_params=pltpu.CompilerParams(use_tc_tiling_on_sc=False),
)(_sc_agg_body)


def _wfull_kernel(c_ref, b_ref, o_ref):
    for r in range(R):
        acc = c_ref[r, 0] * b_ref[0]
        for b in range(1, B):
            acc = acc + c_ref[r, b] * b_ref[b]
        o_ref[r] = acc


def _wfull(coeffs, bases):
    return pl.pallas_call(
        _wfull_kernel,
        in_specs=[
            pl.BlockSpec(memory_space=pltpu.SMEM),
            pl.BlockSpec(memory_space=pltpu.VMEM),
        ],
        out_specs=pl.BlockSpec(memory_space=pltpu.VMEM),
        out_shape=jax.ShapeDtypeStruct((R, H, O), jnp.float32),
    )(coeffs, bases)


MM_BLK = 400


def _chunk_kernel(x_ref, o_ref):
    for hc in range(NCHUNK):
        o_ref[hc] = x_ref[:, hc * HC:(hc + 1) * HC]


def _chunked(x):
    return pl.pallas_call(
        _chunk_kernel,
        grid=(N // MM_BLK,),
        in_specs=[pl.BlockSpec((MM_BLK, H), lambda i: (i, 0))],
        out_specs=pl.BlockSpec((NCHUNK, MM_BLK, HC), lambda i: (0, i, 0)),
        out_shape=jax.ShapeDtypeStruct((NCHUNK, N, HC), jnp.float32),
    )(x)


def _mm_kernel(a_ref, w_ref, o_ref, acc_ref, *, relu, chunked_out):
    r = pl.program_id(1)
    a = jnp.concatenate([a_ref[hc] for hc in range(NCHUNK)], axis=1)
    part = jnp.dot(a, w_ref[0], preferred_element_type=jnp.float32)

    @pl.when(r == 0)
    def _():
        acc_ref[...] = part

    @pl.when(r > 0)
    def _():
        acc_ref[...] = acc_ref[...] + part

    @pl.when(r == R - 1)
    def _():
        res = acc_ref[...]
        if relu:
            res = jnp.maximum(res, 0.0)
        if chunked_out:
            for hc in range(NCHUNK):
                o_ref[hc] = res[:, hc * HC:(hc + 1) * HC]
        else:
            o_ref[...] = res


def _project(agg, w, relu, chunked_out):
    if chunked_out:
        out_specs = pl.BlockSpec((NCHUNK, MM_BLK, HC), lambda i, r: (0, i, 0))
        out_shape = jax.ShapeDtypeStruct((NCHUNK, N, HC), jnp.float32)
    else:
        out_specs = pl.BlockSpec((MM_BLK, O), lambda i, r: (i, 0))
        out_shape = jax.ShapeDtypeStruct((N, O), jnp.float32)
    return pl.pallas_call(
        functools.partial(_mm_kernel, relu=relu, chunked_out=chunked_out),
        grid=(N // MM_BLK, R),
        in_specs=[
            pl.BlockSpec((NCHUNK, MM_BLK, HC),
                         lambda i, r: (0, r * (N // MM_BLK) + i, 0)),
            pl.BlockSpec((1, H, O), lambda i, r: (r, 0, 0)),
        ],
        out_specs=out_specs,
        out_shape=out_shape,
        scratch_shapes=[pltpu.VMEM((MM_BLK, O), jnp.float32)],
    )(agg, w)


@jax.jit
def kernel(emb, bases1, coeffs1, bases2, coeffs2, edge_index, etype, norm):
    src = edge_index[0]
    seg = etype * N + edge_index[1]
    normf = norm.reshape(E)
    zeros = jnp.zeros((ZROWS, HC), jnp.float32)

    xc = _chunked(emb)
    agg1 = _sc_aggregate(xc, src, seg, normf, zeros)
    hc1 = _project(agg1, _wfull(coeffs1, bases1), relu=True, chunked_out=True)
    agg2 = _sc_aggregate(hc1, src, seg, normf, zeros)
    out = _project(agg2, _wfull(coeffs2, bases2), relu=False, chunked_out=False)
    return out


# speedup vs baseline: 1.2879x; 1.2879x over previous
import functools

import jax
import jax.numpy as jnp
from jax import lax
from jax.experimental import pallas as pl
from jax.experimental.pallas import tpu as pltpu
from jax.experimental.pallas import tpu_sc as plsc

N = 10000
E = 160000
H = 256
O = 256
R = 8
B = 4

NSEG = N * R
HC = 16
NCHUNK = H // HC
NC = 2
NS = 16
CPC = NCHUNK // NC
EPT = E // NS
W = 400
NWIN = EPT // W
NPAIR = (NWIN - 1) // 2
SEG_PT = NSEG // NS
TAB_PT = 640
ZROWS = 1250


def _sc_agg_body(xc_hbm, edata_hbm, seg_hbm, zeros_hbm, agg_hbm,
                 table_s, agg_s, zbuf, ed0, ed1, sg0, sg1, rows0, rows1,
                 esem0, esem1, gsem0, gsem1, zsem):
    c = lax.axis_index("c")
    s = lax.axis_index("s")

    pltpu.sync_copy(zeros_hbm, zbuf)

    def start_edata(w, ed, sg, esem):
        base = s * EPT + w * W
        d0 = pltpu.async_copy(edata_hbm.at[:, pl.ds(base, W)], ed, esem)
        d1 = pltpu.async_copy(seg_hbm.at[pl.ds(base, W)], sg, esem)
        return d0, d1

    def scale_rows(ed, rows):
        def scale(g, _):
            nv = plsc.bitcast(ed[1, pl.ds(g * 16, 16)], jnp.float32)
            for j in range(16):
                e = g * 16 + j
                rows[e, :] = rows[e, :] * nv[j]
            return 0

        lax.fori_loop(0, W // 16, scale, 0)

    for i in range(CPC):
        hc = c * CPC + i

        zd = [
            pltpu.async_copy(
                zbuf, agg_s.at[pl.ds(s * SEG_PT + z * ZROWS, ZROWS)], zsem)
            for z in range(SEG_PT // ZROWS)
        ]
        @pl.when(s < NS - 1)
        def _():
            pltpu.async_copy(
                xc_hbm.at[hc, pl.ds(s * TAB_PT, TAB_PT), :],
                table_s.at[pl.ds(s * TAB_PT, TAB_PT)], zsem).wait()

        @pl.when(s == NS - 1)
        def _():
            pltpu.async_copy(
                xc_hbm.at[hc, pl.ds((NS - 1) * TAB_PT, N - (NS - 1) * TAB_PT), :],
                table_s.at[pl.ds((NS - 1) * TAB_PT, N - (NS - 1) * TAB_PT)],
                zsem).wait()

        for d in zd:
            d.wait()
        plsc.subcore_barrier()

        def wait_edata(ed, sg, esem):
            pltpu.make_async_copy(edata_hbm.at[:, pl.ds(0, W)], ed, esem).wait()
            pltpu.make_async_copy(seg_hbm.at[pl.ds(0, W)], sg, esem).wait()

        start_edata(0, ed0, sg0, esem0)
        start_edata(1, ed1, sg1, esem1)

        def pair(g, _):
            a = 2 * g
            wait_edata(ed0, sg0, esem0)
            ga = pltpu.async_copy(table_s.at[ed0.at[0]], rows0, gsem0)
            wait_edata(ed1, sg1, esem1)
            gb = pltpu.async_copy(table_s.at[ed1.at[0]], rows1, gsem1)
            ga.wait()
            scale_rows(ed0, rows0)
            pltpu.sync_copy(rows0, agg_s.at[sg0], add=True)
            start_edata(a + 2, ed0, sg0, esem0)
            gb.wait()
            scale_rows(ed1, rows1)
            pltpu.sync_copy(rows1, agg_s.at[sg1], add=True)

            @pl.when(g < NPAIR - 1)
            def _():
                start_edata(a + 3, ed1, sg1, esem1)

            return 0

        lax.fori_loop(0, NPAIR, pair, 0)
        wait_edata(ed0, sg0, esem0)
        gt = pltpu.async_copy(table_s.at[ed0.at[0]], rows0, gsem0)
        gt.wait()
        scale_rows(ed0, rows0)
        pltpu.sync_copy(rows0, agg_s.at[sg0], add=True)

        plsc.subcore_barrier()

        pltpu.sync_copy(
            agg_s.at[pl.ds(s * SEG_PT, SEG_PT)],
            agg_hbm.at[hc, pl.ds(s * SEG_PT, SEG_PT), :],
        )


_sc_aggregate = functools.partial(
    pl.kernel,
    out_type=jax.ShapeDtypeStruct((NCHUNK, NSEG, HC), jnp.float32),
    mesh=plsc.VectorSubcoreMesh(
        core_axis_name="c", subcore_axis_name="s", num_cores=NC,
        num_subcores=NS),
    scratch_types=[
        pltpu.VMEM_SHARED((N, HC), jnp.float32),
        pltpu.VMEM_SHARED((NSEG, HC), jnp.float32),
        pltpu.VMEM((ZROWS, HC), jnp.float32),
        pltpu.VMEM((2, W), jnp.int32),
        pltpu.VMEM((2, W), jnp.int32),
        pltpu.VMEM((W,), jnp.int32),
        pltpu.VMEM((W,), jnp.int32),
        pltpu.VMEM((W, HC), jnp.float32),
        pltpu.VMEM((W, HC), jnp.float32),
        pltpu.SemaphoreType.DMA,
        pltpu.SemaphoreType.DMA,
        pltpu.SemaphoreType.DMA,
        pltpu.SemaphoreType.DMA,
        pltpu.SemaphoreType.DMA,
    ],
    compiler_params=pltpu.CompilerParams(
        use_tc_tiling_on_sc=False, needs_layout_passes=False),
)(_sc_agg_body)


def _wfull_kernel(c_ref, b_ref, o_ref):
    for r in range(R):
        acc = c_ref[r, 0] * b_ref[0]
        for b in range(1, B):
            acc = acc + c_ref[r, b] * b_ref[b]
        o_ref[r] = acc


def _wfull(coeffs, bases):
    return pl.pallas_call(
        _wfull_kernel,
        in_specs=[
            pl.BlockSpec(memory_space=pltpu.SMEM),
            pl.BlockSpec(memory_space=pltpu.VMEM),
        ],
        out_specs=pl.BlockSpec(memory_space=pltpu.VMEM),
        out_shape=jax.ShapeDtypeStruct((R, H, O), jnp.float32),
    )(coeffs, bases)


MM_BLK = 400


def _chunk_kernel(x_ref, o_ref):
    for hc in range(NCHUNK):
        o_ref[hc] = x_ref[:, hc * HC:(hc + 1) * HC]


def _chunked(x):
    return pl.pallas_call(
        _chunk_kernel,
        grid=(N // MM_BLK,),
        in_specs=[pl.BlockSpec((MM_BLK, H), lambda i: (i, 0))],
        out_specs=pl.BlockSpec((NCHUNK, MM_BLK, HC), lambda i: (0, i, 0)),
        out_shape=jax.ShapeDtypeStruct((NCHUNK, N, HC), jnp.float32),
    )(x)


def _mm_kernel(a_ref, w_ref, o_ref, acc_ref, *, relu, chunked_out):
    r = pl.program_id(1)
    a = jnp.concatenate([a_ref[hc] for hc in range(NCHUNK)], axis=1)
    part = jnp.dot(a, w_ref[0], preferred_element_type=jnp.float32)

    @pl.when(r == 0)
    def _():
        acc_ref[...] = part

    @pl.when(r > 0)
    def _():
        acc_ref[...] = acc_ref[...] + part

    @pl.when(r == R - 1)
    def _():
        res = acc_ref[...]
        if relu:
            res = jnp.maximum(res, 0.0)
        if chunked_out:
            for hc in range(NCHUNK):
                o_ref[hc] = res[:, hc * HC:(hc + 1) * HC]
        else:
            o_ref[...] = res


def _project(agg, w, relu, chunked_out):
    if chunked_out:
        out_specs = pl.BlockSpec((NCHUNK, MM_BLK, HC), lambda i, r: (0, i, 0))
        out_shape = jax.ShapeDtypeStruct((NCHUNK, N, HC), jnp.float32)
    else:
        out_specs = pl.BlockSpec((MM_BLK, O), lambda i, r: (i, 0))
        out_shape = jax.ShapeDtypeStruct((N, O), jnp.float32)
    return pl.pallas_call(
        functools.partial(_mm_kernel, relu=relu, chunked_out=chunked_out),
        grid=(N // MM_BLK, R),
        in_specs=[
            pl.BlockSpec((NCHUNK, MM_BLK, HC),
                         lambda i, r: (0, r * (N // MM_BLK) + i, 0)),
            pl.BlockSpec((1, H, O), lambda i, r: (r, 0, 0)),
        ],
        out_specs=out_specs,
        out_shape=out_shape,
        scratch_shapes=[pltpu.VMEM((MM_BLK, O), jnp.float32)],
    )(agg, w)


@jax.jit
def kernel(emb, bases1, coeffs1, bases2, coeffs2, edge_index, etype, norm):
    src = edge_index[0]
    seg = etype * N + edge_index[1]
    normbits = lax.bitcast_convert_type(norm.reshape(E), jnp.int32)
    edata = jnp.stack([src, normbits])
    zeros = jnp.zeros((ZROWS, HC), jnp.float32)

    xc = _chunked(emb)
    agg1 = _sc_aggregate(xc, edata, seg, zeros)
    hc1 = _project(agg1, _wfull(coeffs1, bases1), relu=True, chunked_out=True)
    agg2 = _sc_aggregate(hc1, edata, seg, zeros)
    out = _project(agg2, _wfull(coeffs2, bases2), relu=False, chunked_out=False)
    return out


# speedup vs baseline: 4.6505x; 3.6108x over previous
import functools

import jax
import jax.numpy as jnp
from jax import lax
from jax.experimental import pallas as pl
from jax.experimental.pallas import tpu as pltpu
from jax.experimental.pallas import tpu_sc as plsc

N = 10000
E = 160000
H = 256
O = 256
R = 8
B = 4

NC = 2
NS = 16
HHALF = 128
EPT = E // NS
W = 80
NWIN = EPT // W
NPAIR = (NWIN - 1) // 2
ROWS_PT = N // NS


def _sc_agg_body(xw_hbm, edata_hbm, dst_hbm, zeros_hbm, out_hbm,
                 acc_s, ed0, ed1, dv0, dv1, rows0, rows1,
                 esem0, esem1, gsem0, gsem1, zsem):
    c = lax.axis_index("c")
    s = lax.axis_index("s")

    zd = pltpu.async_copy(
        zeros_hbm, acc_s.at[pl.ds(s * ROWS_PT, ROWS_PT)], zsem)

    def start_edata(w, ed, dv, esem):
        base = s * EPT + w * W
        d0 = pltpu.async_copy(edata_hbm.at[:, pl.ds(base, W)], ed, esem)
        d1 = pltpu.async_copy(dst_hbm.at[pl.ds(base, W)], dv, esem)
        return d0, d1

    def wait_edata(ed, dv, esem):
        pltpu.make_async_copy(edata_hbm.at[:, pl.ds(0, W)], ed, esem).wait()
        pltpu.make_async_copy(dst_hbm.at[pl.ds(0, W)], dv, esem).wait()

    def scale_rows(ed, rows):
        def scale(g, _):
            nv = plsc.bitcast(ed[1, pl.ds(g * 16, 16)], jnp.float32)
            for j in range(16):
                e = g * 16 + j
                for k in range(HHALF // 16):
                    rows[e, pl.ds(k * 16, 16)] = (
                        rows[e, pl.ds(k * 16, 16)] * nv[j])
            return 0

        lax.fori_loop(0, W // 16, scale, 0)

    start_edata(0, ed0, dv0, esem0)
    start_edata(1, ed1, dv1, esem1)
    zd.wait()
    plsc.subcore_barrier()

    def pair(g, _):
        a = 2 * g
        wait_edata(ed0, dv0, esem0)
        ga = pltpu.async_copy(xw_hbm.at[c].at[ed0.at[0]], rows0, gsem0)
        wait_edata(ed1, dv1, esem1)
        gb = pltpu.async_copy(xw_hbm.at[c].at[ed1.at[0]], rows1, gsem1)
        ga.wait()
        scale_rows(ed0, rows0)
        pltpu.sync_copy(rows0, acc_s.at[dv0], add=True)
        start_edata(a + 2, ed0, dv0, esem0)
        gb.wait()
        scale_rows(ed1, rows1)
        pltpu.sync_copy(rows1, acc_s.at[dv1], add=True)

        @pl.when(g < NPAIR - 1)
        def _():
            start_edata(a + 3, ed1, dv1, esem1)

        return 0

    lax.fori_loop(0, NPAIR, pair, 0)
    wait_edata(ed0, dv0, esem0)
    gt = pltpu.async_copy(xw_hbm.at[c].at[ed0.at[0]], rows0, gsem0)
    gt.wait()
    scale_rows(ed0, rows0)
    pltpu.sync_copy(rows0, acc_s.at[dv0], add=True)

    plsc.subcore_barrier()

    pltpu.sync_copy(
        acc_s.at[pl.ds(s * ROWS_PT, ROWS_PT)],
        out_hbm.at[pl.ds(s * ROWS_PT, ROWS_PT), pl.ds(c * HHALF, HHALF)],
    )


_sc_aggregate = functools.partial(
    pl.kernel,
    out_type=jax.ShapeDtypeStruct((N, H), jnp.float32),
    mesh=plsc.VectorSubcoreMesh(
        core_axis_name="c", subcore_axis_name="s", num_cores=NC,
        num_subcores=NS),
    scratch_types=[
        pltpu.VMEM_SHARED((N, HHALF), jnp.float32),
        pltpu.VMEM((2, W), jnp.int32),
        pltpu.VMEM((2, W), jnp.int32),
        pltpu.VMEM((W,), jnp.int32),
        pltpu.VMEM((W,), jnp.int32),
        pltpu.VMEM((W, HHALF), jnp.float32),
        pltpu.VMEM((W, HHALF), jnp.float32),
        pltpu.SemaphoreType.DMA,
        pltpu.SemaphoreType.DMA,
        pltpu.SemaphoreType.DMA,
        pltpu.SemaphoreType.DMA,
        pltpu.SemaphoreType.DMA,
    ],
    compiler_params=pltpu.CompilerParams(
        use_tc_tiling_on_sc=False, needs_layout_passes=False),
)(_sc_agg_body)


def _wfull_kernel(c_ref, b_ref, o_ref):
    for r in range(R):
        acc = c_ref[r, 0] * b_ref[0]
        for b in range(1, B):
            acc = acc + c_ref[r, b] * b_ref[b]
        o_ref[r] = acc


def _wfull(coeffs, bases):
    return pl.pallas_call(
        _wfull_kernel,
        in_specs=[
            pl.BlockSpec(memory_space=pltpu.SMEM),
            pl.BlockSpec(memory_space=pltpu.VMEM),
        ],
        out_specs=pl.BlockSpec(memory_space=pltpu.VMEM),
        out_shape=jax.ShapeDtypeStruct((R, H, O), jnp.float32),
    )(coeffs, bases)


MM_BLK = 1000


def _xw_kernel(x_ref, w_ref, o_ref, *, relu):
    a = x_ref[...]
    if relu:
        a = jnp.maximum(a, 0.0)
    res = jnp.dot(a, w_ref[0], preferred_element_type=jnp.float32)
    o_ref[0] = res[:, :HHALF]
    o_ref[1] = res[:, HHALF:]


def _xw(x, w, relu):
    nb = N // MM_BLK
    return pl.pallas_call(
        functools.partial(_xw_kernel, relu=relu),
        grid=(nb, R),
        in_specs=[
            pl.BlockSpec((MM_BLK, H), lambda i, r: (i, 0)),
            pl.BlockSpec((1, H, O), lambda i, r: (r, 0, 0)),
        ],
        out_specs=pl.BlockSpec((NC, MM_BLK, HHALF),
                               lambda i, r: (0, r * nb + i, 0)),
        out_shape=jax.ShapeDtypeStruct((NC, R * N, HHALF), jnp.float32),
    )(x, w)


@jax.jit
def kernel(emb, bases1, coeffs1, bases2, coeffs2, edge_index, etype, norm):
    erow = etype * N + edge_index[0]
    normbits = lax.bitcast_convert_type(norm.reshape(E), jnp.int32)
    edata = jnp.stack([erow, normbits])
    dst = edge_index[1]
    zeros = jnp.zeros((ROWS_PT, HHALF), jnp.float32)

    xw1 = _xw(emb, _wfull(coeffs1, bases1), relu=False)
    h = _sc_aggregate(xw1, edata, dst, zeros)
    xw2 = _xw(h, _wfull(coeffs2, bases2), relu=True)
    return _sc_aggregate(xw2, edata, dst, zeros)


# speedup vs baseline: 5.2422x; 1.1273x over previous
import functools

import jax
import jax.numpy as jnp
from jax import lax
from jax.experimental import pallas as pl
from jax.experimental.pallas import tpu as pltpu
from jax.experimental.pallas import tpu_sc as plsc

N = 10000
E = 160000
H = 256
O = 256
R = 8
B = 4

NC = 2
NS = 16
HHALF = 128
EPT = E // NS
W = 80
NWIN = EPT // W
NPAIR = (NWIN - 1) // 2
ROWS_PT = N // NS


def _sc_agg_body(xw_hbm, edata_hbm, dst_hbm, zeros_hbm, out_hbm,
                 acc_s, ed0, ed1, dv0, dv1, rows0, rows1,
                 esem0, esem1, gsem0, gsem1, zsem, ssem0, ssem1):
    c = lax.axis_index("c")
    s = lax.axis_index("s")

    zd = pltpu.async_copy(
        zeros_hbm, acc_s.at[pl.ds(s * ROWS_PT, ROWS_PT)], zsem)

    def start_edata(w, ed, dv, esem):
        base = s * EPT + w * W
        d0 = pltpu.async_copy(edata_hbm.at[:, pl.ds(base, W)], ed, esem)
        d1 = pltpu.async_copy(dst_hbm.at[pl.ds(base, W)], dv, esem)
        return d0, d1

    def wait_edata(ed, dv, esem):
        pltpu.make_async_copy(edata_hbm.at[:, pl.ds(0, W)], ed, esem).wait()
        pltpu.make_async_copy(dst_hbm.at[pl.ds(0, W)], dv, esem).wait()

    def scale_rows(ed, rows):
        def scale(g, _):
            nv = plsc.bitcast(ed[1, pl.ds(g * 16, 16)], jnp.float32)
            for j in range(16):
                e = g * 16 + j
                for k in range(HHALF // 16):
                    rows[e, pl.ds(k * 16, 16)] = (
                        rows[e, pl.ds(k * 16, 16)] * nv[j])
            return 0

        lax.fori_loop(0, W // 16, scale, 0)

    start_edata(0, ed0, dv0, esem0)
    start_edata(1, ed1, dv1, esem1)
    zd.wait()
    plsc.subcore_barrier()

    def wait_scatter(rows, dv, ssem):
        pltpu.make_async_copy(rows, acc_s.at[dv], ssem).wait()

    def pair(g, _):
        a = 2 * g
        wait_edata(ed0, dv0, esem0)

        @pl.when(g > 0)
        def _():
            wait_scatter(rows0, dv0, ssem0)

        ga = pltpu.async_copy(xw_hbm.at[c].at[ed0.at[0]], rows0, gsem0)
        wait_edata(ed1, dv1, esem1)

        @pl.when(g > 0)
        def _():
            wait_scatter(rows1, dv1, ssem1)

        gb = pltpu.async_copy(xw_hbm.at[c].at[ed1.at[0]], rows1, gsem1)
        ga.wait()
        scale_rows(ed0, rows0)
        pltpu.async_copy(rows0, acc_s.at[dv0], ssem0, add=True)
        start_edata(a + 2, ed0, dv0, esem0)
        gb.wait()
        scale_rows(ed1, rows1)
        pltpu.async_copy(rows1, acc_s.at[dv1], ssem1, add=True)

        @pl.when(g < NPAIR - 1)
        def _():
            start_edata(a + 3, ed1, dv1, esem1)

        return 0

    lax.fori_loop(0, NPAIR, pair, 0)
    wait_edata(ed0, dv0, esem0)
    wait_scatter(rows0, dv0, ssem0)
    gt = pltpu.async_copy(xw_hbm.at[c].at[ed0.at[0]], rows0, gsem0)
    gt.wait()
    scale_rows(ed0, rows0)
    wait_scatter(rows1, dv1, ssem1)
    pltpu.sync_copy(rows0, acc_s.at[dv0], add=True)

    plsc.subcore_barrier()

    pltpu.sync_copy(
        acc_s.at[pl.ds(s * ROWS_PT, ROWS_PT)],
        out_hbm.at[pl.ds(s * ROWS_PT, ROWS_PT), pl.ds(c * HHALF, HHALF)],
    )


_sc_aggregate = functools.partial(
    pl.kernel,
    out_type=jax.ShapeDtypeStruct((N, H), jnp.float32),
    mesh=plsc.VectorSubcoreMesh(
        core_axis_name="c", subcore_axis_name="s", num_cores=NC,
        num_subcores=NS),
    scratch_types=[
        pltpu.VMEM_SHARED((N, HHALF), jnp.float32),
        pltpu.VMEM((2, W), jnp.int32),
        pltpu.VMEM((2, W), jnp.int32),
        pltpu.VMEM((W,), jnp.int32),
        pltpu.VMEM((W,), jnp.int32),
        pltpu.VMEM((W, HHALF), jnp.float32),
        pltpu.VMEM((W, HHALF), jnp.float32),
        pltpu.SemaphoreType.DMA,
        pltpu.SemaphoreType.DMA,
        pltpu.SemaphoreType.DMA,
        pltpu.SemaphoreType.DMA,
        pltpu.SemaphoreType.DMA,
        pltpu.SemaphoreType.DMA,
        pltpu.SemaphoreType.DMA,
    ],
    compiler_params=pltpu.CompilerParams(
        use_tc_tiling_on_sc=False, needs_layout_passes=False),
)(_sc_agg_body)


def _wfull_kernel(c_ref, b_ref, o_ref):
    for r in range(R):
        acc = c_ref[r, 0] * b_ref[0]
        for b in range(1, B):
            acc = acc + c_ref[r, b] * b_ref[b]
        o_ref[r] = acc


def _wfull(coeffs, bases):
    return pl.pallas_call(
        _wfull_kernel,
        in_specs=[
            pl.BlockSpec(memory_space=pltpu.SMEM),
            pl.BlockSpec(memory_space=pltpu.VMEM),
        ],
        out_specs=pl.BlockSpec(memory_space=pltpu.VMEM),
        out_shape=jax.ShapeDtypeStruct((R, H, O), jnp.float32),
    )(coeffs, bases)


MM_BLK = 1000


def _xw_kernel(x_ref, w_ref, o_ref, *, relu):
    a = x_ref[...]
    if relu:
        a = jnp.maximum(a, 0.0)
    res = jnp.dot(a, w_ref[0], preferred_element_type=jnp.float32)
    o_ref[0] = res[:, :HHALF]
    o_ref[1] = res[:, HHALF:]


def _xw(x, w, relu):
    nb = N // MM_BLK
    return pl.pallas_call(
        functools.partial(_xw_kernel, relu=relu),
        grid=(nb, R),
        in_specs=[
            pl.BlockSpec((MM_BLK, H), lambda i, r: (i, 0)),
            pl.BlockSpec((1, H, O), lambda i, r: (r, 0, 0)),
        ],
        out_specs=pl.BlockSpec((NC, MM_BLK, HHALF),
                               lambda i, r: (0, r * nb + i, 0)),
        out_shape=jax.ShapeDtypeStruct((NC, R * N, HHALF), jnp.float32),
    )(x, w)


@jax.jit
def kernel(emb, bases1, coeffs1, bases2, coeffs2, edge_index, etype, norm):
    erow = etype * N + edge_index[0]
    normbits = lax.bitcast_convert_type(norm.reshape(E), jnp.int32)
    edata = jnp.stack([erow, normbits])
    dst = edge_index[1]
    zeros = jnp.zeros((ROWS_PT, HHALF), jnp.float32)

    xw1 = _xw(emb, _wfull(coeffs1, bases1), relu=False)
    h = _sc_aggregate(xw1, edata, dst, zeros)
    xw2 = _xw(h, _wfull(coeffs2, bases2), relu=True)
    return _sc_aggregate(xw2, edata, dst, zeros)


# speedup vs baseline: 5.8087x; 1.1081x over previous
import functools

import jax
import jax.numpy as jnp
from jax import lax
from jax.experimental import pallas as pl
from jax.experimental.pallas import tpu as pltpu
from jax.experimental.pallas import tpu_sc as plsc

N = 10000
E = 160000
H = 256
O = 256
R = 8
B = 4

NC = 2
NS = 16
HHALF = 128
EPT = E // NS
W = 80
NWIN = EPT // W
NPAIR = (NWIN - 1) // 2
ROWS_PT = N // NS


def _sc_agg_body(xw_hbm, edata_hbm, dst_hbm, zeros_hbm, out_hbm,
                 acc_s, ed0, ed1, dv0, dv1, rows0, rows1,
                 esem0, esem1, gsem0, gsem1, zsem, ssem0, ssem1):
    c = lax.axis_index("c")
    s = lax.axis_index("s")

    zd = pltpu.async_copy(
        zeros_hbm, acc_s.at[pl.ds(s * ROWS_PT, ROWS_PT)], zsem)

    def start_edata(w, ed, dv, esem):
        base = s * EPT + w * W
        d0 = pltpu.async_copy(edata_hbm.at[:, pl.ds(base, W)], ed, esem)
        d1 = pltpu.async_copy(dst_hbm.at[pl.ds(base, W)], dv, esem)
        return d0, d1

    def wait_edata(ed, dv, esem):
        pltpu.make_async_copy(edata_hbm.at[:, pl.ds(0, W)], ed, esem).wait()
        pltpu.make_async_copy(dst_hbm.at[pl.ds(0, W)], dv, esem).wait()

    def scale_rows(ed, rows):
        def scale(g, _):
            nv = plsc.bitcast(ed[1, pl.ds(g * 16, 16)], jnp.float32)
            for j in range(16):
                e = g * 16 + j
                for k in range(HHALF // 16):
                    rows[e, pl.ds(k * 16, 16)] = (
                        rows[e, pl.ds(k * 16, 16)] * nv[j])
            return 0

        lax.fori_loop(0, W // 16, scale, 0)

    start_edata(0, ed0, dv0, esem0)
    start_edata(1, ed1, dv1, esem1)
    zd.wait()
    plsc.subcore_barrier()

    def wait_scatter(rows, dv, ssem):
        pltpu.make_async_copy(rows, acc_s.at[dv], ssem).wait()

    def pair(g, _):
        a = 2 * g
        wait_edata(ed0, dv0, esem0)

        @pl.when(g > 0)
        def _():
            wait_scatter(rows0, dv0, ssem0)

        ga = pltpu.async_copy(xw_hbm.at[c].at[ed0.at[0]], rows0, gsem0)
        wait_edata(ed1, dv1, esem1)

        @pl.when(g > 0)
        def _():
            wait_scatter(rows1, dv1, ssem1)

        gb = pltpu.async_copy(xw_hbm.at[c].at[ed1.at[0]], rows1, gsem1)
        ga.wait()
        scale_rows(ed0, rows0)
        pltpu.async_copy(rows0, acc_s.at[dv0], ssem0, add=True)
        start_edata(a + 2, ed0, dv0, esem0)
        gb.wait()
        scale_rows(ed1, rows1)
        pltpu.async_copy(rows1, acc_s.at[dv1], ssem1, add=True)

        @pl.when(g < NPAIR - 1)
        def _():
            start_edata(a + 3, ed1, dv1, esem1)

        return 0

    lax.fori_loop(0, NPAIR, pair, 0)
    wait_edata(ed0, dv0, esem0)
    wait_scatter(rows0, dv0, ssem0)
    gt = pltpu.async_copy(xw_hbm.at[c].at[ed0.at[0]], rows0, gsem0)
    gt.wait()
    scale_rows(ed0, rows0)
    wait_scatter(rows1, dv1, ssem1)
    pltpu.sync_copy(rows0, acc_s.at[dv0], add=True)

    plsc.subcore_barrier()

    pltpu.sync_copy(
        acc_s.at[pl.ds(s * ROWS_PT, ROWS_PT)],
        out_hbm.at[pl.ds(s * ROWS_PT, ROWS_PT), pl.ds(c * HHALF, HHALF)],
    )


_sc_aggregate = functools.partial(
    pl.kernel,
    out_type=jax.ShapeDtypeStruct((N, H), jnp.float32),
    mesh=plsc.VectorSubcoreMesh(
        core_axis_name="c", subcore_axis_name="s", num_cores=NC,
        num_subcores=NS),
    scratch_types=[
        pltpu.VMEM_SHARED((N, HHALF), jnp.float32),
        pltpu.VMEM((2, W), jnp.int32),
        pltpu.VMEM((2, W), jnp.int32),
        pltpu.VMEM((W,), jnp.int32),
        pltpu.VMEM((W,), jnp.int32),
        pltpu.VMEM((W, HHALF), jnp.float32),
        pltpu.VMEM((W, HHALF), jnp.float32),
        pltpu.SemaphoreType.DMA,
        pltpu.SemaphoreType.DMA,
        pltpu.SemaphoreType.DMA,
        pltpu.SemaphoreType.DMA,
        pltpu.SemaphoreType.DMA,
        pltpu.SemaphoreType.DMA,
        pltpu.SemaphoreType.DMA,
    ],
    compiler_params=pltpu.CompilerParams(
        use_tc_tiling_on_sc=False, needs_layout_passes=False),
)(_sc_agg_body)


def _wfull_kernel(c_ref, b_ref, o_ref):
    for r in range(R):
        acc = c_ref[r, 0] * b_ref[0]
        for b in range(1, B):
            acc = acc + c_ref[r, b] * b_ref[b]
        o_ref[r] = acc


def _wfull(coeffs, bases):
    return pl.pallas_call(
        _wfull_kernel,
        in_specs=[
            pl.BlockSpec(memory_space=pltpu.SMEM),
            pl.BlockSpec(memory_space=pltpu.VMEM),
        ],
        out_specs=pl.BlockSpec(memory_space=pltpu.VMEM),
        out_shape=jax.ShapeDtypeStruct((R, H, O), jnp.float32),
    )(coeffs, bases)


MM_BLK = 2000


def _xw_kernel(x_ref, w_ref, o_ref, *, relu):
    a = x_ref[...]
    if relu:
        a = jnp.maximum(a, 0.0)
    res = jnp.dot(a, w_ref[0], preferred_element_type=jnp.float32)
    o_ref[0] = res[:, :HHALF]
    o_ref[1] = res[:, HHALF:]


def _xw(x, w, relu):
    nb = N // MM_BLK
    return pl.pallas_call(
        functools.partial(_xw_kernel, relu=relu),
        grid=(nb, R),
        in_specs=[
            pl.BlockSpec((MM_BLK, H), lambda i, r: (i, 0)),
            pl.BlockSpec((1, H, O), lambda i, r: (r, 0, 0)),
        ],
        out_specs=pl.BlockSpec((NC, MM_BLK, HHALF),
                               lambda i, r: (0, r * nb + i, 0)),
        out_shape=jax.ShapeDtypeStruct((NC, R * N, HHALF), jnp.float32),
    )(x, w)


@jax.jit
def kernel(emb, bases1, coeffs1, bases2, coeffs2, edge_index, etype, norm):
    erow = etype * N + edge_index[0]
    normbits = lax.bitcast_convert_type(norm.reshape(E), jnp.int32)
    edata = jnp.stack([erow, normbits])
    dst = edge_index[1]
    zeros = jnp.zeros((ROWS_PT, HHALF), jnp.float32)

    xw1 = _xw(emb, _wfull(coeffs1, bases1), relu=False)
    h = _sc_aggregate(xw1, edata, dst, zeros)
    xw2 = _xw(h, _wfull(coeffs2, bases2), relu=True)
    return _sc_aggregate(xw2, edata, dst, zeros)


# speedup vs baseline: 6.2072x; 1.0686x over previous
import functools

import jax
import jax.numpy as jnp
from jax import lax
from jax.experimental import pallas as pl
from jax.experimental.pallas import tpu as pltpu
from jax.experimental.pallas import tpu_sc as plsc

N = 10000
E = 160000
H = 256
O = 256
R = 8
B = 4

NC = 2
NS = 16
HHALF = 128
EPT = E // NS
W = 80
NWIN = EPT // W
NPAIR = (NWIN - 1) // 2
ROWS_PT = N // NS


def _sc_agg_body(xw_hbm, edata_hbm, dst_hbm, zeros_hbm, out_hbm,
                 acc_s, ed0, ed1, dv0, dv1, rows0, rows1,
                 esem0, esem1, gsem0, gsem1, zsem, ssem0, ssem1):
    c = lax.axis_index("c")
    s = lax.axis_index("s")

    zd = pltpu.async_copy(
        zeros_hbm, acc_s.at[pl.ds(s * ROWS_PT, ROWS_PT)], zsem)

    def start_edata(w, ed, dv, esem):
        base = s * EPT + w * W
        d0 = pltpu.async_copy(edata_hbm.at[:, pl.ds(base, W)], ed, esem)
        d1 = pltpu.async_copy(dst_hbm.at[pl.ds(base, W)], dv, esem)
        return d0, d1

    def wait_edata(ed, dv, esem):
        pltpu.make_async_copy(edata_hbm.at[:, pl.ds(0, W)], ed, esem).wait()
        pltpu.make_async_copy(dst_hbm.at[pl.ds(0, W)], dv, esem).wait()

    def scale_rows(ed, rows):
        def scale(g, _):
            nv = plsc.bitcast(ed[1, pl.ds(g * 16, 16)], jnp.float32)
            for j in range(16):
                e = g * 16 + j
                for k in range(HHALF // 16):
                    rows[e, pl.ds(k * 16, 16)] = (
                        rows[e, pl.ds(k * 16, 16)] * nv[j])
            return 0

        lax.fori_loop(0, W // 16, scale, 0)

    start_edata(0, ed0, dv0, esem0)
    start_edata(1, ed1, dv1, esem1)
    zd.wait()
    plsc.subcore_barrier()

    def wait_scatter(rows, dv, ssem):
        pltpu.make_async_copy(rows, acc_s.at[dv], ssem).wait()

    def pair(g, _):
        a = 2 * g
        wait_edata(ed0, dv0, esem0)

        @pl.when(g > 0)
        def _():
            wait_scatter(rows0, dv0, ssem0)

        ga = pltpu.async_copy(xw_hbm.at[c].at[ed0.at[0]], rows0, gsem0)
        wait_edata(ed1, dv1, esem1)

        @pl.when(g > 0)
        def _():
            wait_scatter(rows1, dv1, ssem1)

        gb = pltpu.async_copy(xw_hbm.at[c].at[ed1.at[0]], rows1, gsem1)
        ga.wait()
        scale_rows(ed0, rows0)
        pltpu.async_copy(rows0, acc_s.at[dv0], ssem0, add=True)
        start_edata(a + 2, ed0, dv0, esem0)
        gb.wait()
        scale_rows(ed1, rows1)
        pltpu.async_copy(rows1, acc_s.at[dv1], ssem1, add=True)

        @pl.when(g < NPAIR - 1)
        def _():
            start_edata(a + 3, ed1, dv1, esem1)

        return 0

    lax.fori_loop(0, NPAIR, pair, 0)
    wait_edata(ed0, dv0, esem0)
    wait_scatter(rows0, dv0, ssem0)
    gt = pltpu.async_copy(xw_hbm.at[c].at[ed0.at[0]], rows0, gsem0)
    gt.wait()
    scale_rows(ed0, rows0)
    wait_scatter(rows1, dv1, ssem1)
    pltpu.sync_copy(rows0, acc_s.at[dv0], add=True)

    plsc.subcore_barrier()

    pltpu.sync_copy(
        acc_s.at[pl.ds(s * ROWS_PT, ROWS_PT)],
        out_hbm.at[pl.ds(s * ROWS_PT, ROWS_PT), pl.ds(c * HHALF, HHALF)],
    )


_sc_aggregate = functools.partial(
    pl.kernel,
    out_type=jax.ShapeDtypeStruct((N, H), jnp.float32),
    mesh=plsc.VectorSubcoreMesh(
        core_axis_name="c", subcore_axis_name="s", num_cores=NC,
        num_subcores=NS),
    scratch_types=[
        pltpu.VMEM_SHARED((N, HHALF), jnp.float32),
        pltpu.VMEM((2, W), jnp.int32),
        pltpu.VMEM((2, W), jnp.int32),
        pltpu.VMEM((W,), jnp.int32),
        pltpu.VMEM((W,), jnp.int32),
        pltpu.VMEM((W, HHALF), jnp.float32),
        pltpu.VMEM((W, HHALF), jnp.float32),
        pltpu.SemaphoreType.DMA,
        pltpu.SemaphoreType.DMA,
        pltpu.SemaphoreType.DMA,
        pltpu.SemaphoreType.DMA,
        pltpu.SemaphoreType.DMA,
        pltpu.SemaphoreType.DMA,
        pltpu.SemaphoreType.DMA,
    ],
    compiler_params=pltpu.CompilerParams(
        use_tc_tiling_on_sc=False, needs_layout_passes=False),
)(_sc_agg_body)


def _wfull_kernel(c_ref, b_ref, o_ref):
    for r in range(R):
        acc = c_ref[r, 0] * b_ref[0]
        for b in range(1, B):
            acc = acc + c_ref[r, b] * b_ref[b]
        o_ref[r] = acc


def _wfull(coeffs, bases):
    return pl.pallas_call(
        _wfull_kernel,
        in_specs=[
            pl.BlockSpec(memory_space=pltpu.SMEM),
            pl.BlockSpec(memory_space=pltpu.VMEM),
        ],
        out_specs=pl.BlockSpec(memory_space=pltpu.VMEM),
        out_shape=jax.ShapeDtypeStruct((R, H, O), jnp.float32),
    )(coeffs, bases)


MM_BLK = 5000


def _xw_kernel(x_ref, w_ref, o_ref, *, relu):
    a = x_ref[...]
    if relu:
        a = jnp.maximum(a, 0.0)
    res = jnp.dot(a, w_ref[0], preferred_element_type=jnp.float32)
    o_ref[0] = res[:, :HHALF]
    o_ref[1] = res[:, HHALF:]


def _xw(x, w, relu):
    nb = N // MM_BLK
    return pl.pallas_call(
        functools.partial(_xw_kernel, relu=relu),
        grid=(nb, R),
        in_specs=[
            pl.BlockSpec((MM_BLK, H), lambda i, r: (i, 0)),
            pl.BlockSpec((1, H, O), lambda i, r: (r, 0, 0)),
        ],
        out_specs=pl.BlockSpec((NC, MM_BLK, HHALF),
                               lambda i, r: (0, r * nb + i, 0)),
        out_shape=jax.ShapeDtypeStruct((NC, R * N, HHALF), jnp.float32),
    )(x, w)


@jax.jit
def kernel(emb, bases1, coeffs1, bases2, coeffs2, edge_index, etype, norm):
    erow = etype * N + edge_index[0]
    normbits = lax.bitcast_convert_type(norm.reshape(E), jnp.int32)
    edata = jnp.stack([erow, normbits])
    dst = edge_index[1]
    zeros = jnp.zeros((ROWS_PT, HHALF), jnp.float32)

    xw1 = _xw(emb, _wfull(coeffs1, bases1), relu=False)
    h = _sc_aggregate(xw1, edata, dst, zeros)
    xw2 = _xw(h, _wfull(coeffs2, bases2), relu=True)
    return _sc_aggregate(xw2, edata, dst, zeros)


# speedup vs baseline: 6.3263x; 1.0192x over previous
import functools

import jax
import jax.numpy as jnp
from jax import lax
from jax.experimental import pallas as pl
from jax.experimental.pallas import tpu as pltpu
from jax.experimental.pallas import tpu_sc as plsc

N = 10000
E = 160000
H = 256
O = 256
R = 8
B = 4

NC = 2
NS = 16
HHALF = 128
EPT = E // NS
W = 80
NWIN = EPT // W
NPAIR = (NWIN - 1) // 2
ROWS_PT = N // NS


def _sc_agg_body(xw_hbm, edata_hbm, dst_hbm, zeros_hbm, out_hbm,
                 acc_s, ed0, ed1, dv0, dv1, rows0, rows1,
                 esem0, esem1, gsem0, gsem1, zsem, ssem0, ssem1):
    c = lax.axis_index("c")
    s = lax.axis_index("s")

    zd = pltpu.async_copy(
        zeros_hbm, acc_s.at[pl.ds(s * ROWS_PT, ROWS_PT)], zsem)

    def start_edata(w, ed, dv, esem):
        base = s * EPT + w * W
        d0 = pltpu.async_copy(edata_hbm.at[:, pl.ds(base, W)], ed, esem)
        d1 = pltpu.async_copy(dst_hbm.at[pl.ds(base, W)], dv, esem)
        return d0, d1

    def wait_edata(ed, dv, esem):
        pltpu.make_async_copy(edata_hbm.at[:, pl.ds(0, W)], ed, esem).wait()
        pltpu.make_async_copy(dst_hbm.at[pl.ds(0, W)], dv, esem).wait()

    def scale_rows(ed, rows):
        def scale(g, _):
            nv = plsc.bitcast(ed[1, pl.ds(g * 16, 16)], jnp.float32)
            for j in range(16):
                e = g * 16 + j
                for k in range(HHALF // 16):
                    rows[e, pl.ds(k * 16, 16)] = (
                        rows[e, pl.ds(k * 16, 16)] * nv[j])
            return 0

        lax.fori_loop(0, W // 16, scale, 0)

    start_edata(0, ed0, dv0, esem0)
    start_edata(1, ed1, dv1, esem1)
    zd.wait()
    plsc.subcore_barrier()

    def wait_scatter(rows, dv, ssem):
        pltpu.make_async_copy(rows, acc_s.at[dv], ssem).wait()

    def pair(g, _):
        a = 2 * g
        wait_edata(ed0, dv0, esem0)

        @pl.when(g > 0)
        def _():
            wait_scatter(rows0, dv0, ssem0)

        ga = pltpu.async_copy(xw_hbm.at[c].at[ed0.at[0]], rows0, gsem0)
        wait_edata(ed1, dv1, esem1)

        @pl.when(g > 0)
        def _():
            wait_scatter(rows1, dv1, ssem1)

        gb = pltpu.async_copy(xw_hbm.at[c].at[ed1.at[0]], rows1, gsem1)
        ga.wait()
        scale_rows(ed0, rows0)
        pltpu.async_copy(rows0, acc_s.at[dv0], ssem0, add=True)
        start_edata(a + 2, ed0, dv0, esem0)
        gb.wait()
        scale_rows(ed1, rows1)
        pltpu.async_copy(rows1, acc_s.at[dv1], ssem1, add=True)

        @pl.when(g < NPAIR - 1)
        def _():
            start_edata(a + 3, ed1, dv1, esem1)

        return 0

    lax.fori_loop(0, NPAIR, pair, 0)
    wait_edata(ed0, dv0, esem0)
    wait_scatter(rows0, dv0, ssem0)
    gt = pltpu.async_copy(xw_hbm.at[c].at[ed0.at[0]], rows0, gsem0)
    gt.wait()
    scale_rows(ed0, rows0)
    wait_scatter(rows1, dv1, ssem1)
    pltpu.sync_copy(rows0, acc_s.at[dv0], add=True)

    plsc.subcore_barrier()

    pltpu.sync_copy(
        acc_s.at[pl.ds(s * ROWS_PT, ROWS_PT)],
        out_hbm.at[pl.ds(s * ROWS_PT, ROWS_PT), pl.ds(c * HHALF, HHALF)],
    )


_sc_aggregate = functools.partial(
    pl.kernel,
    out_type=jax.ShapeDtypeStruct((N, H), jnp.float32),
    mesh=plsc.VectorSubcoreMesh(
        core_axis_name="c", subcore_axis_name="s", num_cores=NC,
        num_subcores=NS),
    scratch_types=[
        pltpu.VMEM_SHARED((N, HHALF), jnp.float32),
        pltpu.VMEM((2, W), jnp.int32),
        pltpu.VMEM((2, W), jnp.int32),
        pltpu.VMEM((W,), jnp.int32),
        pltpu.VMEM((W,), jnp.int32),
        pltpu.VMEM((W, HHALF), jnp.float32),
        pltpu.VMEM((W, HHALF), jnp.float32),
        pltpu.SemaphoreType.DMA,
        pltpu.SemaphoreType.DMA,
        pltpu.SemaphoreType.DMA,
        pltpu.SemaphoreType.DMA,
        pltpu.SemaphoreType.DMA,
        pltpu.SemaphoreType.DMA,
        pltpu.SemaphoreType.DMA,
    ],
    compiler_params=pltpu.CompilerParams(
        use_tc_tiling_on_sc=False, needs_layout_passes=False),
)(_sc_agg_body)


def _wfull_kernel(c_ref, b_ref, o_ref):
    for r in range(R):
        acc = c_ref[r, 0] * b_ref[0]
        for b in range(1, B):
            acc = acc + c_ref[r, b] * b_ref[b]
        o_ref[r] = acc


def _wfull(coeffs, bases):
    return pl.pallas_call(
        _wfull_kernel,
        in_specs=[
            pl.BlockSpec(memory_space=pltpu.SMEM),
            pl.BlockSpec(memory_space=pltpu.VMEM),
        ],
        out_specs=pl.BlockSpec(memory_space=pltpu.VMEM),
        out_shape=jax.ShapeDtypeStruct((R, H, O), jnp.float32),
    )(coeffs, bases)


MM_BLK = 10000


def _xw_kernel(x_ref, w_ref, o_ref, *, relu):
    a = x_ref[...]
    if relu:
        a = jnp.maximum(a, 0.0)
    res = jnp.dot(a, w_ref[0], preferred_element_type=jnp.float32)
    o_ref[0] = res[:, :HHALF]
    o_ref[1] = res[:, HHALF:]


def _xw(x, w, relu):
    nb = N // MM_BLK
    return pl.pallas_call(
        functools.partial(_xw_kernel, relu=relu),
        grid=(nb, R),
        in_specs=[
            pl.BlockSpec((MM_BLK, H), lambda i, r: (i, 0)),
            pl.BlockSpec((1, H, O), lambda i, r: (r, 0, 0)),
        ],
        out_specs=pl.BlockSpec((NC, MM_BLK, HHALF),
                               lambda i, r: (0, r * nb + i, 0)),
        out_shape=jax.ShapeDtypeStruct((NC, R * N, HHALF), jnp.float32),
    )(x, w)


@jax.jit
def kernel(emb, bases1, coeffs1, bases2, coeffs2, edge_index, etype, norm):
    erow = etype * N + edge_index[0]
    normbits = lax.bitcast_convert_type(norm.reshape(E), jnp.int32)
    edata = jnp.stack([erow, normbits])
    dst = edge_index[1]
    zeros = jnp.zeros((ROWS_PT, HHALF), jnp.float32)

    xw1 = _xw(emb, _wfull(coeffs1, bases1), relu=False)
    h = _sc_aggregate(xw1, edata, dst, zeros)
    xw2 = _xw(h, _wfull(coeffs2, bases2), relu=True)
    return _sc_aggregate(xw2, edata, dst, zeros)
